# baseline (speedup 1.0000x reference)
import functools

import jax
import jax.numpy as jnp
from jax import lax
from jax.experimental import pallas as pl
from jax.experimental.pallas import tpu as pltpu
from jax.experimental.pallas import tpu_sc as plsc

N_TOK = 16384
N_EMBD = 2048
N_EXPERTS = 16
TOP_K = 2

TILE = 1024

_info = plsc.get_sparse_core_info()
_NC, _NS = _info.num_cores, _info.num_subcores
_NW = _NC * _NS
_TPW = N_TOK // _NW


def _logits_krn(x_ref, w_ref, b_ref, epst_ref, noisyt_ref):
    x = x_ref[...]
    y = jnp.dot(x, w_ref[...], preferred_element_type=jnp.float32)
    yt = y.T + b_ref[...]
    logits = yt[:N_EXPERTS, :]
    nlog = yt[N_EXPERTS:, :]
    noisyt_ref[...] = logits + epst_ref[...] * jax.nn.softplus(nlog)


def _route_sc_krn(noisy_hbm, routt_hbm, i1_hbm, i2_hbm, g1t_hbm,
                  nz_v, routt_v, i1_v, i2_v, g1t_v):
    wid = lax.axis_index("s") * _NC + lax.axis_index("c")
    base = wid * _TPW
    for e in range(N_EXPERTS):
        pltpu.sync_copy(noisy_hbm.at[e, pl.ds(base, _TPW)],
                        nz_v.at[pl.ds(e * _TPW, _TPW)])

    def body(tb, carry):
        t0 = tb * 16
        r = [nz_v[pl.ds(e * _TPW + t0, 16)] for e in range(N_EXPERTS)]
        m1 = r[0]
        for e in range(1, N_EXPERTS):
            m1 = jnp.maximum(m1, r[e])
        i1 = jnp.full((16,), N_EXPERTS - 1, jnp.int32)
        for e in range(N_EXPERTS - 2, -1, -1):
            i1 = jnp.where(r[e] == m1, e, i1)
        m2 = None
        for e in range(N_EXPERTS):
            masked = jnp.where(i1 == e, -jnp.inf, r[e])
            m2 = masked if m2 is None else jnp.maximum(m2, masked)
        i2 = jnp.full((16,), N_EXPERTS - 1, jnp.int32)
        for e in range(N_EXPERTS - 2, -1, -1):
            i2 = jnp.where((r[e] == m2) & (i1 != e), e, i2)
        ex = []
        s = None
        for e in range(N_EXPERTS):
            t = jnp.exp(r[e] - m1)
            ex.append(t)
            s = t if s is None else s + t
        rs = 1.0 / s
        d = 1.0 / (1.0 + jnp.exp(m2 - m1))
        for e in range(N_EXPERTS):
            g1t_v[pl.ds(e * _TPW + t0, 16)] = ex[e] * rs
            keep = (i1 == e) | (i2 == e)
            routt_v[pl.ds(e * _TPW + t0, 16)] = jnp.where(keep, ex[e] * d, 0.0)
        i1_v[pl.ds(t0, 16)] = i1
        i2_v[pl.ds(t0, 16)] = i2
        return carry

    lax.fori_loop(0, _TPW // 16, body, 0)

    for e in range(N_EXPERTS):
        pltpu.sync_copy(g1t_v.at[pl.ds(e * _TPW, _TPW)],
                        g1t_hbm.at[e, pl.ds(base, _TPW)])
        pltpu.sync_copy(routt_v.at[pl.ds(e * _TPW, _TPW)],
                        routt_hbm.at[e, pl.ds(base, _TPW)])
    pltpu.sync_copy(i1_v, i1_hbm.at[pl.ds(base, _TPW)])
    pltpu.sync_copy(i2_v, i2_hbm.at[pl.ds(base, _TPW)])


@functools.partial(jax.jit, static_argnames=())
def kernel(mh_output, W_route, b_route, W_noise, b_noise, noise_eps):
    grid = (N_TOK // TILE,)
    W = jnp.concatenate([W_route, W_noise], axis=1)
    b = jnp.concatenate([b_route, b_noise]).reshape(2 * N_EXPERTS, 1)
    epst = noise_eps.T
    noisyt = pl.pallas_call(
        _logits_krn,
        grid=grid,
        compiler_params=pltpu.CompilerParams(
            dimension_semantics=("parallel",)),
        in_specs=[
            pl.BlockSpec((TILE, N_EMBD), lambda i: (i, 0)),
            pl.BlockSpec((N_EMBD, 2 * N_EXPERTS), lambda i: (0, 0)),
            pl.BlockSpec((2 * N_EXPERTS, 1), lambda i: (0, 0)),
            pl.BlockSpec((N_EXPERTS, TILE), lambda i: (0, i)),
        ],
        out_specs=pl.BlockSpec((N_EXPERTS, TILE), lambda i: (0, i)),
        out_shape=jax.ShapeDtypeStruct((N_EXPERTS, N_TOK), jnp.float32),
    )(mh_output, W, b, epst)

    route = pl.kernel(
        _route_sc_krn,
        mesh=plsc.VectorSubcoreMesh(core_axis_name="c", subcore_axis_name="s"),
        out_type=[
            jax.ShapeDtypeStruct((N_EXPERTS, N_TOK), jnp.float32),
            jax.ShapeDtypeStruct((N_TOK,), jnp.int32),
            jax.ShapeDtypeStruct((N_TOK,), jnp.int32),
            jax.ShapeDtypeStruct((N_EXPERTS, N_TOK), jnp.float32),
        ],
        scratch_types=[
            pltpu.VMEM((N_EXPERTS * _TPW,), jnp.float32),
            pltpu.VMEM((N_EXPERTS * _TPW,), jnp.float32),
            pltpu.VMEM((_TPW,), jnp.int32),
            pltpu.VMEM((_TPW,), jnp.int32),
            pltpu.VMEM((N_EXPERTS * _TPW,), jnp.float32),
        ],
    )
    routt, i1, i2, g1t = route(noisyt)
    return (routt.T, jnp.stack([i1, i2], axis=1), g1t.T)


# speedup vs baseline: 1.6262x; 1.6262x over previous
import functools

import jax
import jax.numpy as jnp
from jax.experimental import pallas as pl
from jax.experimental.pallas import tpu as pltpu

N_TOK = 16384
N_EMBD = 2048
N_EXPERTS = 16
TOP_K = 2

TILE = 1024


def _router_krn(x_ref, w_ref, b_ref, epst_ref, routt_ref, idxt_ref, g1t_ref):
    x = x_ref[...]
    y = jnp.dot(x, w_ref[...], preferred_element_type=jnp.float32)
    yt = y.T + b_ref[...]
    logits = yt[:N_EXPERTS, :]
    nlog = yt[N_EXPERTS:, :]
    noisy = logits + epst_ref[...] * jax.nn.softplus(nlog)

    m1 = jnp.max(noisy, axis=0, keepdims=True)
    e_all = jnp.exp(noisy - m1)
    g1t_ref[...] = e_all / jnp.sum(e_all, axis=0, keepdims=True)

    lane = jax.lax.broadcasted_iota(jnp.int32, noisy.shape, 0)
    big = jnp.int32(N_EXPERTS)
    i1 = jnp.min(jnp.where(noisy == m1, lane, big), axis=0, keepdims=True)
    rest = jnp.where(lane == i1, -jnp.inf, noisy)
    m2 = jnp.max(rest, axis=0, keepdims=True)
    i2 = jnp.min(jnp.where(rest == m2, lane, big), axis=0, keepdims=True)
    idxt_ref[...] = jnp.concatenate([i1, i2], axis=0)

    keep = (lane == i1) | (lane == i2)
    routt_ref[...] = jnp.where(keep, e_all, 0.0) / (1.0 + jnp.exp(m2 - m1))


@functools.partial(jax.jit, static_argnames=())
def kernel(mh_output, W_route, b_route, W_noise, b_noise, noise_eps):
    grid = (N_TOK // TILE,)
    W = jnp.concatenate([W_route, W_noise], axis=1)
    b = jnp.concatenate([b_route, b_noise]).reshape(2 * N_EXPERTS, 1)
    epst = noise_eps.T
    routt, idxt, g1t = pl.pallas_call(
        _router_krn,
        grid=grid,
        compiler_params=pltpu.CompilerParams(
            dimension_semantics=("parallel",)),
        in_specs=[
            pl.BlockSpec((TILE, N_EMBD), lambda i: (i, 0)),
            pl.BlockSpec((N_EMBD, 2 * N_EXPERTS), lambda i: (0, 0)),
            pl.BlockSpec((2 * N_EXPERTS, 1), lambda i: (0, 0)),
            pl.BlockSpec((N_EXPERTS, TILE), lambda i: (0, i)),
        ],
        out_specs=[
            pl.BlockSpec((N_EXPERTS, TILE), lambda i: (0, i)),
            pl.BlockSpec((TOP_K, TILE), lambda i: (0, i)),
            pl.BlockSpec((N_EXPERTS, TILE), lambda i: (0, i)),
        ],
        out_shape=[
            jax.ShapeDtypeStruct((N_EXPERTS, N_TOK), jnp.float32),
            jax.ShapeDtypeStruct((TOP_K, N_TOK), jnp.int32),
            jax.ShapeDtypeStruct((N_EXPERTS, N_TOK), jnp.float32),
        ],
    )(mh_output, W, b, epst)
    return (routt.T, idxt.T, g1t.T)
